# Initial kernel scaffold; baseline (speedup 1.0000x reference)
import jax
import jax.numpy as jnp
from jax.experimental import pallas as pl


def kernel(x, params, scopes, idx):
    raise NotImplementedError("write your pallas kernel here")


# speedup vs baseline: 2.1395x; 2.1395x over previous
import math

import jax
import jax.numpy as jnp
from jax import lax
from jax.experimental import pallas as pl
from jax.experimental.pallas import tpu as pltpu

_N_NODES = 50000
_N_VARS = 128
_BATCH = 1024
_W = 512
_N_BLOCKS = (_N_NODES + _W - 1) // _W
_N_PAD = _N_BLOCKS * _W
_HALF_LOG_2PI = 0.5 * math.log(2.0 * math.pi)
_BIG = 1e30


def _tc_body(x_ref, meta_ref, out_ref):
    xb = x_ref[...]
    xb = jnp.where(jnp.isnan(xb), jnp.float32(_BIG), xb)
    mu = meta_ref[0:1, :]
    sigma = jnp.maximum(meta_ref[1:2, :], jnp.float32(1e-5))
    scope = meta_ref[2:3, :].astype(jnp.int32)
    iot = lax.broadcasted_iota(jnp.int32, (_N_VARS, _W), 0)
    oneh = jnp.where(iot == scope, jnp.float32(1.0), jnp.float32(0.0))
    val = jnp.dot(xb, oneh, preferred_element_type=jnp.float32)
    z = (val - mu) / sigma
    lld = -0.5 * z * z - jnp.log(sigma) - _HALF_LOG_2PI
    out_ref[...] = jnp.where(val >= jnp.float32(_BIG * 0.5),
                             jnp.float32(0.0), lld)


def _tc_logprob(x, meta_t):
    return pl.pallas_call(
        _tc_body,
        grid=(_N_BLOCKS,),
        in_specs=[
            pl.BlockSpec((_BATCH, _N_VARS), lambda i: (0, 0)),
            pl.BlockSpec((8, _W), lambda i: (0, i)),
        ],
        out_specs=pl.BlockSpec((_BATCH, _W), lambda i: (0, i)),
        out_shape=jax.ShapeDtypeStruct((_BATCH, _N_NODES), jnp.float32),
        compiler_params=pltpu.CompilerParams(
            dimension_semantics=("arbitrary",)),
    )(x, meta_t)


def kernel(x, params, scopes, idx):
    mu = params[:, 0]
    sg = params[:, 1]
    sc = scopes.astype(jnp.float32)
    src = jnp.stack([mu, sg, sc])
    meta3 = jnp.zeros((3, _N_PAD), jnp.float32).at[:, idx].set(src)
    meta_t = jnp.concatenate(
        [meta3, jnp.zeros((5, _N_PAD), jnp.float32)], axis=0)
    return _tc_logprob(x, meta_t)


# speedup vs baseline: 2.2019x; 1.0292x over previous
import functools
import math

import jax
import jax.numpy as jnp
from jax import lax
from jax.experimental import pallas as pl
from jax.experimental.pallas import tpu as pltpu
from jax.experimental.pallas import tpu_sc as plsc

_N_NODES = 50000
_N_VARS = 128
_BATCH = 1024
_HALF_LOG_2PI = 0.5 * math.log(2.0 * math.pi)
_BIG = 1e30

_NC, _NS = 2, 16
_NW = _NC * _NS
_CHUNKS = 13
_NODES_PER_W = _CHUNKS * 128
_N_PAD = _NW * _NODES_PER_W
_R = 16

_W = 4096
_N_BLOCKS = (_N_NODES + _W - 1) // _W


def _sc_permute_body(idx_hbm, meta_hbm, mu_hbm, sg_hbm, sc_hbm,
                     idx_v, meta_v, sem):
    c = lax.axis_index("c")
    s = lax.axis_index("s")
    wid = s * _NC + c
    pltpu.sync_copy(idx_hbm.at[wid], idx_v)
    pltpu.sync_copy(meta_hbm.at[wid], meta_v)
    copies = []
    for k, out in enumerate((mu_hbm, sg_hbm, sc_hbm)):
        for j in range(_CHUNKS):
            copies.append(
                pltpu.async_copy(meta_v.at[k, j], out.at[idx_v.at[j]], sem))
    for cp in copies:
        cp.wait()


def _sc_permute(idx3, meta4):
    f = pl.kernel(
        _sc_permute_body,
        out_type=(jax.ShapeDtypeStruct((_N_PAD,), jnp.float32),) * 3,
        mesh=plsc.VectorSubcoreMesh(
            core_axis_name="c", subcore_axis_name="s"),
        scratch_types=[
            pltpu.VMEM((_CHUNKS, 128), jnp.int32),
            pltpu.VMEM((3, _CHUNKS, 128), jnp.float32),
            pltpu.SemaphoreType.DMA,
        ],
    )
    return f(idx3, meta4)


def _tc_body(x_ref, mu_ref, sg_ref, sc_ref, out_ref):
    xb = x_ref[...]
    xb = jnp.where(jnp.isnan(xb), jnp.float32(_BIG), xb)
    mu = mu_ref[...]
    sigma = jnp.maximum(sg_ref[...], jnp.float32(1e-5))
    scope = sc_ref[...].astype(jnp.int32)
    iot = lax.broadcasted_iota(jnp.int32, (_N_VARS, _W), 0)
    oneh = jnp.where(iot == scope, jnp.float32(1.0), jnp.float32(0.0))
    val = jnp.dot(xb, oneh, preferred_element_type=jnp.float32)
    z = (val - mu) / sigma
    lld = -0.5 * z * z - jnp.log(sigma) - _HALF_LOG_2PI
    out_ref[...] = jnp.where(val >= jnp.float32(_BIG * 0.5),
                             jnp.float32(0.0), lld)


def _tc_logprob(x, mu_p, sg_p, sc_p):
    row = pl.BlockSpec((1, _W), lambda i: (0, i))
    return pl.pallas_call(
        _tc_body,
        grid=(_N_BLOCKS,),
        in_specs=[
            pl.BlockSpec((_BATCH, _N_VARS), lambda i: (0, 0)),
            row, row, row,
        ],
        out_specs=pl.BlockSpec((_BATCH, _W), lambda i: (0, i)),
        out_shape=jax.ShapeDtypeStruct((_BATCH, _N_NODES), jnp.float32),
        compiler_params=pltpu.CompilerParams(
            dimension_semantics=("arbitrary",)),
    )(x, mu_p.reshape(1, _N_PAD), sg_p.reshape(1, _N_PAD),
      sc_p.reshape(1, _N_PAD))


def kernel(x, params, scopes, idx):
    mu = params[:, 0]
    sg = params[:, 1]
    sc = scopes.astype(jnp.float32)
    idx_pad = jnp.concatenate(
        [idx, jnp.arange(_N_NODES, _N_PAD, dtype=jnp.int32)])
    idx3 = idx_pad.reshape(_NW, _CHUNKS, 128)
    src3 = jnp.stack([mu, sg, sc])
    meta = jnp.pad(src3, ((0, 0), (0, _N_PAD - _N_NODES)))
    meta4 = meta.reshape(3, _NW, _CHUNKS, 128).transpose(1, 0, 2, 3)
    mu_p, sg_p, sc_p = _sc_permute(idx3, meta4)
    return _tc_logprob(x, mu_p, sg_p, sc_p)


# speedup vs baseline: 3.2539x; 1.4778x over previous
import math

import jax
import jax.numpy as jnp
from jax import lax
from jax.experimental import pallas as pl
from jax.experimental.pallas import tpu as pltpu
from jax.experimental.pallas import tpu_sc as plsc

_N_NODES = 50000
_N_VARS = 128
_BATCH = 1024
_HALF_LOG_2PI = 0.5 * math.log(2.0 * math.pi)
_BIG = 1e30

_NC, _NS = 2, 16
_OWN = 3328
_N_PAD = _NS * _OWN
_HALF = _OWN // _NC
_CHUNK = 4096
_N_CHUNKS = _N_PAD // _CHUNK
_STEPS = _CHUNK // 16

_W = 4096
_N_BLOCKS = (_N_NODES + _W - 1) // _W


def _sc_permute_body(idx_hbm, mu_hbm, sg_hbm, sc_hbm,
                     mu_out, sg_out, sc_out,
                     idx_v, mu_v, sg_v, sc_v,
                     mu_loc, sg_loc, sc_loc, sem):
    c = lax.axis_index("c")
    s = lax.axis_index("s")
    lo = s * _OWN
    hi = lo + _OWN

    def stage(k, buf):
        return pltpu.async_copy(
            idx_hbm.at[pl.ds(k * _CHUNK, _CHUNK)], idx_v.at[buf], sem), \
            pltpu.async_copy(
                mu_hbm.at[pl.ds(k * _CHUNK, _CHUNK)], mu_v.at[buf], sem), \
            pltpu.async_copy(
                sg_hbm.at[pl.ds(k * _CHUNK, _CHUNK)], sg_v.at[buf], sem), \
            pltpu.async_copy(
                sc_hbm.at[pl.ds(k * _CHUNK, _CHUNK)], sc_v.at[buf], sem)

    pending = stage(0, 0)
    for k in range(_N_CHUNKS):
        for cp in pending:
            cp.wait()
        buf = k % 2
        if k + 1 < _N_CHUNKS:
            pending = stage(k + 1, (k + 1) % 2)

        def step(i, _):
            sl = pl.ds(i * 16, 16)
            iv = idx_v.at[buf][sl]
            pos = iv - lo
            m = (iv >= lo) & (iv < hi)
            plsc.store_scatter(mu_loc, [pos], mu_v.at[buf][sl], mask=m)
            plsc.store_scatter(sg_loc, [pos], sg_v.at[buf][sl], mask=m)
            plsc.store_scatter(sc_loc, [pos], sc_v.at[buf][sl], mask=m)
            return 0

        lax.fori_loop(0, _STEPS, step, 0)

    base_loc = c * _HALF
    base_out = lo + base_loc
    pltpu.sync_copy(mu_loc.at[pl.ds(base_loc, _HALF)],
                    mu_out.at[pl.ds(base_out, _HALF)])
    pltpu.sync_copy(sg_loc.at[pl.ds(base_loc, _HALF)],
                    sg_out.at[pl.ds(base_out, _HALF)])
    pltpu.sync_copy(sc_loc.at[pl.ds(base_loc, _HALF)],
                    sc_out.at[pl.ds(base_out, _HALF)])


def _sc_permute(idx_pad, mu_pad, sg_pad, sc_pad):
    f = pl.kernel(
        _sc_permute_body,
        out_type=(jax.ShapeDtypeStruct((_N_PAD,), jnp.float32),) * 3,
        mesh=plsc.VectorSubcoreMesh(
            core_axis_name="c", subcore_axis_name="s"),
        compiler_params=pltpu.CompilerParams(
            use_tc_tiling_on_sc=False, needs_layout_passes=False),
        scratch_types=[
            pltpu.VMEM((2, _CHUNK), jnp.int32),
            pltpu.VMEM((2, _CHUNK), jnp.float32),
            pltpu.VMEM((2, _CHUNK), jnp.float32),
            pltpu.VMEM((2, _CHUNK), jnp.float32),
            pltpu.VMEM((_OWN,), jnp.float32),
            pltpu.VMEM((_OWN,), jnp.float32),
            pltpu.VMEM((_OWN,), jnp.float32),
            pltpu.SemaphoreType.DMA,
        ],
    )
    return f(idx_pad, mu_pad, sg_pad, sc_pad)


def _tc_body(x_ref, mu_ref, sg_ref, sc_ref, out_ref):
    xb = x_ref[...]
    xb = jnp.where(jnp.isnan(xb), jnp.float32(_BIG), xb)
    mu = mu_ref[...]
    sigma = jnp.maximum(sg_ref[...], jnp.float32(1e-5))
    scope = sc_ref[...].astype(jnp.int32)
    iot = lax.broadcasted_iota(jnp.int32, (_N_VARS, _W), 0)
    oneh = jnp.where(iot == scope, jnp.float32(1.0), jnp.float32(0.0))
    val = jnp.dot(xb, oneh, preferred_element_type=jnp.float32)
    z = (val - mu) / sigma
    lld = -0.5 * z * z - jnp.log(sigma) - _HALF_LOG_2PI
    out_ref[...] = jnp.where(val >= jnp.float32(_BIG * 0.5),
                             jnp.float32(0.0), lld)


def _tc_logprob(x, mu_p, sg_p, sc_p):
    row = pl.BlockSpec((1, _W), lambda i: (0, i))
    return pl.pallas_call(
        _tc_body,
        grid=(_N_BLOCKS,),
        in_specs=[
            pl.BlockSpec((_BATCH, _N_VARS), lambda i: (0, 0)),
            row, row, row,
        ],
        out_specs=pl.BlockSpec((_BATCH, _W), lambda i: (0, i)),
        out_shape=jax.ShapeDtypeStruct((_BATCH, _N_NODES), jnp.float32),
        compiler_params=pltpu.CompilerParams(
            dimension_semantics=("arbitrary",)),
    )(x, mu_p.reshape(1, _N_PAD), sg_p.reshape(1, _N_PAD),
      sc_p.reshape(1, _N_PAD))


def kernel(x, params, scopes, idx):
    pad = _N_PAD - _N_NODES
    idx_pad = jnp.concatenate(
        [idx, jnp.arange(_N_NODES, _N_PAD, dtype=jnp.int32)])
    mu_pad = jnp.pad(params[:, 0], (0, pad))
    sg_pad = jnp.pad(params[:, 1], (0, pad))
    sc_pad = jnp.pad(scopes.astype(jnp.float32), (0, pad))
    mu_p, sg_p, sc_p = _sc_permute(idx_pad, mu_pad, sg_pad, sc_pad)
    return _tc_logprob(x, mu_p, sg_p, sc_p)


# speedup vs baseline: 3.4905x; 1.0727x over previous
import math

import jax
import jax.numpy as jnp
from jax import lax
from jax.experimental import pallas as pl
from jax.experimental.pallas import tpu as pltpu
from jax.experimental.pallas import tpu_sc as plsc

_N_NODES = 50000
_N_VARS = 128
_BATCH = 1024
_HALF_LOG_2PI = 0.5 * math.log(2.0 * math.pi)
_BIG = 1e30

_NC, _NS = 2, 16
_OWN = 3328
_N_PAD = _NS * _OWN
_HALF = _OWN // _NC
_CHUNK = 4096
_N_CHUNKS = _N_PAD // _CHUNK
_STEPS = _CHUNK // 16

_W = 4096
_N_BLOCKS = (_N_NODES + _W - 1) // _W


def _sc_permute_body(idx_hbm, mu_hbm, sg_hbm, sc_hbm,
                     mu_out, sg_out, sc_out,
                     idx_v, mu_v, sg_v, sc_v,
                     mu_loc, sg_loc, sc_loc, sem):
    c = lax.axis_index("c")
    s = lax.axis_index("s")
    lo = s * _OWN
    hi = lo + _OWN

    def stage(k, buf):
        return pltpu.async_copy(
            idx_hbm.at[pl.ds(k * _CHUNK, _CHUNK)], idx_v.at[buf], sem), \
            pltpu.async_copy(
                mu_hbm.at[pl.ds(k * _CHUNK, _CHUNK)], mu_v.at[buf], sem), \
            pltpu.async_copy(
                sg_hbm.at[pl.ds(k * _CHUNK, _CHUNK)], sg_v.at[buf], sem), \
            pltpu.async_copy(
                sc_hbm.at[pl.ds(k * _CHUNK, _CHUNK)], sc_v.at[buf], sem)

    pending = stage(0, 0)
    for k in range(_N_CHUNKS):
        for cp in pending:
            cp.wait()
        buf = k % 2
        if k + 1 < _N_CHUNKS:
            pending = stage(k + 1, (k + 1) % 2)

        @plsc.parallel_loop(0, _STEPS, 1, unroll=8)
        def _chunk_scan(i):
            sl = pl.ds(i * 16, 16)
            iv = idx_v.at[buf][sl]
            pos = iv - lo
            m = (iv >= lo) & (iv < hi)
            plsc.store_scatter(mu_loc, [pos], mu_v.at[buf][sl], mask=m)
            plsc.store_scatter(sg_loc, [pos], sg_v.at[buf][sl], mask=m)
            plsc.store_scatter(sc_loc, [pos], sc_v.at[buf][sl], mask=m)

    base_loc = c * _HALF
    base_out = lo + base_loc
    pltpu.sync_copy(mu_loc.at[pl.ds(base_loc, _HALF)],
                    mu_out.at[pl.ds(base_out, _HALF)])
    pltpu.sync_copy(sg_loc.at[pl.ds(base_loc, _HALF)],
                    sg_out.at[pl.ds(base_out, _HALF)])
    pltpu.sync_copy(sc_loc.at[pl.ds(base_loc, _HALF)],
                    sc_out.at[pl.ds(base_out, _HALF)])


def _sc_permute(idx_pad, mu_pad, sg_pad, sc_pad):
    f = pl.kernel(
        _sc_permute_body,
        out_type=(jax.ShapeDtypeStruct((_N_PAD,), jnp.float32),) * 3,
        mesh=plsc.VectorSubcoreMesh(
            core_axis_name="c", subcore_axis_name="s"),
        compiler_params=pltpu.CompilerParams(
            use_tc_tiling_on_sc=False, needs_layout_passes=False),
        scratch_types=[
            pltpu.VMEM((2, _CHUNK), jnp.int32),
            pltpu.VMEM((2, _CHUNK), jnp.float32),
            pltpu.VMEM((2, _CHUNK), jnp.float32),
            pltpu.VMEM((2, _CHUNK), jnp.float32),
            pltpu.VMEM((_OWN,), jnp.float32),
            pltpu.VMEM((_OWN,), jnp.float32),
            pltpu.VMEM((_OWN,), jnp.float32),
            pltpu.SemaphoreType.DMA,
        ],
    )
    return f(idx_pad, mu_pad, sg_pad, sc_pad)


def _tc_body(x_ref, mu_ref, sg_ref, sc_ref, out_ref):
    xb = x_ref[...]
    xb = jnp.where(jnp.isnan(xb), jnp.float32(_BIG), xb)
    mu = mu_ref[...]
    sigma = jnp.maximum(sg_ref[...], jnp.float32(1e-5))
    scope = sc_ref[...].astype(jnp.int32)
    iot = lax.broadcasted_iota(jnp.int32, (_N_VARS, _W), 0)
    oneh = jnp.where(iot == scope, jnp.float32(1.0), jnp.float32(0.0))
    val = jnp.dot(xb, oneh, preferred_element_type=jnp.float32)
    z = (val - mu) / sigma
    lld = -0.5 * z * z - jnp.log(sigma) - _HALF_LOG_2PI
    out_ref[...] = jnp.where(val >= jnp.float32(_BIG * 0.5),
                             jnp.float32(0.0), lld)


def _tc_logprob(x, mu_p, sg_p, sc_p):
    row = pl.BlockSpec((1, _W), lambda i: (0, i))
    return pl.pallas_call(
        _tc_body,
        grid=(_N_BLOCKS,),
        in_specs=[
            pl.BlockSpec((_BATCH, _N_VARS), lambda i: (0, 0)),
            row, row, row,
        ],
        out_specs=pl.BlockSpec((_BATCH, _W), lambda i: (0, i)),
        out_shape=jax.ShapeDtypeStruct((_BATCH, _N_NODES), jnp.float32),
        compiler_params=pltpu.CompilerParams(
            dimension_semantics=("arbitrary",)),
    )(x, mu_p.reshape(1, _N_PAD), sg_p.reshape(1, _N_PAD),
      sc_p.reshape(1, _N_PAD))


def kernel(x, params, scopes, idx):
    pad = _N_PAD - _N_NODES
    idx_pad = jnp.concatenate(
        [idx, jnp.arange(_N_NODES, _N_PAD, dtype=jnp.int32)])
    mu_pad = jnp.pad(params[:, 0], (0, pad))
    sg_pad = jnp.pad(params[:, 1], (0, pad))
    sc_pad = jnp.pad(scopes.astype(jnp.float32), (0, pad))
    mu_p, sg_p, sc_p = _sc_permute(idx_pad, mu_pad, sg_pad, sc_pad)
    return _tc_logprob(x, mu_p, sg_p, sc_p)
